# baseline (speedup 1.0000x reference)
import jax
import jax.numpy as jnp
from jax.experimental import pallas as pl

N_ROWS = 64
N_COLS = 32768
ROW_BLOCK = 32


def _hardmax_block(x_ref, o_ref):
    xb = x_ref[...]
    idx = jnp.argmax(xb, axis=1)
    o_ref[...] = jnp.zeros(o_ref.shape, jnp.int32)
    lane_iota = jax.lax.broadcasted_iota(jnp.int32, (1, 128), 1)
    for r in range(ROW_BLOCK):
        c = idx[r]
        base = (c // 128) * 128
        vec = (lane_iota == (c - base)).astype(jnp.int32)
        o_ref[pl.ds(r, 1), pl.ds(base, 128)] = vec


def kernel(x):
    return pl.pallas_call(
        _hardmax_block,
        grid=(N_ROWS // ROW_BLOCK,),
        in_specs=[pl.BlockSpec((ROW_BLOCK, N_COLS), lambda i: (i, 0))],
        out_specs=pl.BlockSpec((ROW_BLOCK, N_COLS), lambda i: (i, 0)),
        out_shape=jax.ShapeDtypeStruct((N_ROWS, N_COLS), jnp.int32),
    )(x)
